# Initial kernel scaffold; baseline (speedup 1.0000x reference)
import jax
import jax.numpy as jnp
from jax.experimental import pallas as pl


def kernel(x, edge_index, W_self0, W_neigh0, b0, W_self1, W_neigh1, b1, W_self2, W_neigh2, b2):
    raise NotImplementedError("write your pallas kernel here")


# speedup vs baseline: 4.1964x; 4.1964x over previous
import functools

import jax
import jax.numpy as jnp
from jax import lax
from jax.experimental import pallas as pl
from jax.experimental.pallas import tpu as pltpu
from jax.experimental.pallas import tpu_sc as plsc

N = 10000
E = 320000
D_IN = 128
D_HID = 128
D_OUT = 47
D_OUT_PAD = 128

NC = 2
NS = 16
NW = NC * NS
CHUNK = 128

CH_PER_W = -(-E // (NW * CHUNK))
E_PER_W = CH_PER_W * CHUNK
E_PAD = NW * E_PER_W

ACC_ROWS = 10240
ZROWS_PER_TILE = ACC_ROWS // NS

DEG_W = 128


def _make_agg(width):
    mesh = plsc.VectorSubcoreMesh(core_axis_name="c", subcore_axis_name="s")

    @functools.partial(
        pl.kernel,
        out_type=jax.ShapeDtypeStruct((NC * ACC_ROWS, width), jnp.float32),
        mesh=mesh,
        scratch_types=[
            pltpu.VMEM((CHUNK,), jnp.int32),
            pltpu.VMEM((CHUNK,), jnp.int32),
            pltpu.VMEM((CHUNK, width), jnp.float32),
            pltpu.VMEM_SHARED((ACC_ROWS, width), jnp.float32),
            pltpu.SemaphoreType.DMA,
        ],
    )
    def agg(p_hbm, src_hbm, dst_hbm, zeros_hbm, out_hbm,
            src_v, dst_v, rows_v, acc, sem):
        c = lax.axis_index("c")
        s = lax.axis_index("s")

        pltpu.sync_copy(zeros_hbm, rows_v)
        z0 = s * ZROWS_PER_TILE
        for j in range(ZROWS_PER_TILE // CHUNK):
            pltpu.sync_copy(rows_v, acc.at[pl.ds(z0 + j * CHUNK, CHUNK)])
        plsc.subcore_barrier()

        base = (c * NS + s) * E_PER_W

        def body(g, carry):
            eb = base + g * CHUNK
            pltpu.sync_copy(src_hbm.at[pl.ds(eb, CHUNK)], src_v)
            pltpu.sync_copy(dst_hbm.at[pl.ds(eb, CHUNK)], dst_v)
            pltpu.async_copy(p_hbm.at[src_v], rows_v, sem).wait()
            pltpu.sync_copy(rows_v, acc.at[dst_v], add=True)
            return carry

        lax.fori_loop(0, CH_PER_W, body, 0)
        plsc.subcore_barrier()

        for j in range(ZROWS_PER_TILE // CHUNK):
            pltpu.sync_copy(acc.at[pl.ds(z0 + j * CHUNK, CHUNK)], rows_v)
            pltpu.sync_copy(rows_v,
                            out_hbm.at[pl.ds(c * ACC_ROWS + z0 + j * CHUNK,
                                             CHUNK)])

    return agg


def _make_deg():
    mesh = plsc.VectorSubcoreMesh(core_axis_name="c", subcore_axis_name="s")

    @functools.partial(
        pl.kernel,
        out_type=jax.ShapeDtypeStruct((NC * ACC_ROWS, DEG_W), jnp.float32),
        mesh=mesh,
        scratch_types=[
            pltpu.VMEM((CHUNK,), jnp.int32),
            pltpu.VMEM((CHUNK, DEG_W), jnp.float32),
            pltpu.VMEM((CHUNK, DEG_W), jnp.float32),
            pltpu.VMEM_SHARED((ACC_ROWS, DEG_W), jnp.float32),
        ],
    )
    def deg(ones_hbm, zeros_hbm, dst_hbm, out_hbm,
            dst_v, ones_v, zeros_v, acc):
        c = lax.axis_index("c")
        s = lax.axis_index("s")

        pltpu.sync_copy(zeros_hbm, zeros_v)
        pltpu.sync_copy(ones_hbm, ones_v)
        z0 = s * ZROWS_PER_TILE
        for j in range(ZROWS_PER_TILE // CHUNK):
            pltpu.sync_copy(zeros_v, acc.at[pl.ds(z0 + j * CHUNK, CHUNK)])
        plsc.subcore_barrier()

        base = (c * NS + s) * E_PER_W

        def body(g, carry):
            eb = base + g * CHUNK
            pltpu.sync_copy(dst_hbm.at[pl.ds(eb, CHUNK)], dst_v)
            pltpu.sync_copy(ones_v, acc.at[dst_v], add=True)
            return carry

        lax.fori_loop(0, CH_PER_W, body, 0)
        plsc.subcore_barrier()

        for j in range(ZROWS_PER_TILE // CHUNK):
            pltpu.sync_copy(acc.at[pl.ds(z0 + j * CHUNK, CHUNK)], ones_v)
            pltpu.sync_copy(ones_v,
                            out_hbm.at[pl.ds(c * ACC_ROWS + z0 + j * CHUNK,
                                             CHUNK)])

    return deg


_BN = 2000


def _tc_first_body(x_ref, wn_ref, ws_ref, b_ref, p_ref, s_ref):
    xb = x_ref[...]
    p_ref[...] = jnp.dot(xb, wn_ref[...], preferred_element_type=jnp.float32)
    s_ref[...] = (jnp.dot(xb, ws_ref[...], preferred_element_type=jnp.float32)
                  + b_ref[...])


def _tc_first(x, wn, ws, b):
    return pl.pallas_call(
        _tc_first_body,
        grid=(N // _BN,),
        in_specs=[
            pl.BlockSpec((_BN, D_IN), lambda i: (i, 0)),
            pl.BlockSpec((D_IN, D_HID), lambda i: (0, 0)),
            pl.BlockSpec((D_IN, D_HID), lambda i: (0, 0)),
            pl.BlockSpec((1, D_HID), lambda i: (0, 0)),
        ],
        out_specs=[
            pl.BlockSpec((_BN, D_HID), lambda i: (i, 0)),
            pl.BlockSpec((_BN, D_HID), lambda i: (i, 0)),
        ],
        out_shape=[
            jax.ShapeDtypeStruct((N, D_HID), jnp.float32),
            jax.ShapeDtypeStruct((N, D_HID), jnp.float32),
        ],
    )(x, wn, ws, b)


def _tc_mid_body(s_ref, agg_ref, deg_ref, wn_ref, ws_ref, b_ref,
                 p_ref, so_ref):
    a = agg_ref[0] + agg_ref[1]
    dcnt = deg_ref[0, :, 0:1] + deg_ref[1, :, 0:1]
    d = jnp.maximum(dcnt, 1.0)
    h = jnp.maximum(s_ref[...] + a / d, 0.0)
    p_ref[...] = jnp.dot(h, wn_ref[...], preferred_element_type=jnp.float32)
    so_ref[...] = (jnp.dot(h, ws_ref[...], preferred_element_type=jnp.float32)
                   + b_ref[...])


def _tc_mid(s_prev, agg, deg, wn, ws, b, d_out):
    return pl.pallas_call(
        _tc_mid_body,
        grid=(N // _BN,),
        in_specs=[
            pl.BlockSpec((_BN, D_HID), lambda i: (i, 0)),
            pl.BlockSpec((NC, _BN, D_HID), lambda i: (0, i, 0)),
            pl.BlockSpec((NC, _BN, DEG_W), lambda i: (0, i, 0)),
            pl.BlockSpec((D_HID, d_out), lambda i: (0, 0)),
            pl.BlockSpec((D_HID, d_out), lambda i: (0, 0)),
            pl.BlockSpec((1, d_out), lambda i: (0, 0)),
        ],
        out_specs=[
            pl.BlockSpec((_BN, d_out), lambda i: (i, 0)),
            pl.BlockSpec((_BN, d_out), lambda i: (i, 0)),
        ],
        out_shape=[
            jax.ShapeDtypeStruct((N, d_out), jnp.float32),
            jax.ShapeDtypeStruct((N, d_out), jnp.float32),
        ],
    )(s_prev, agg, deg, wn, ws, b)


def _tc_last_body(s_ref, agg_ref, deg_ref, o_ref):
    a = agg_ref[0] + agg_ref[1]
    dcnt = deg_ref[0, :, 0:1] + deg_ref[1, :, 0:1]
    d = jnp.maximum(dcnt, 1.0)
    o_ref[...] = s_ref[...] + a / d


def _tc_last(s_prev, agg, deg):
    return pl.pallas_call(
        _tc_last_body,
        grid=(N // _BN,),
        in_specs=[
            pl.BlockSpec((_BN, D_OUT_PAD), lambda i: (i, 0)),
            pl.BlockSpec((NC, _BN, D_OUT_PAD), lambda i: (0, i, 0)),
            pl.BlockSpec((NC, _BN, DEG_W), lambda i: (0, i, 0)),
        ],
        out_specs=pl.BlockSpec((_BN, D_OUT_PAD), lambda i: (i, 0)),
        out_shape=jax.ShapeDtypeStruct((N, D_OUT_PAD), jnp.float32),
    )(s_prev, agg, deg)


def kernel(x, edge_index, W_self0, W_neigh0, b0,
           W_self1, W_neigh1, b1, W_self2, W_neigh2, b2):
    src = edge_index[0].astype(jnp.int32)
    dst = edge_index[1].astype(jnp.int32)
    pad = E_PAD - E
    src_p = jnp.concatenate([src, jnp.zeros((pad,), jnp.int32)])
    dst_p = jnp.concatenate([dst, jnp.full((pad,), N, jnp.int32)])

    zeros128 = jnp.zeros((CHUNK, D_HID), jnp.float32)
    zeros64 = jnp.zeros((CHUNK, D_OUT_PAD), jnp.float32)
    ones128 = jnp.ones((CHUNK, DEG_W), jnp.float32)

    wn2 = jnp.pad(W_neigh2, ((0, 0), (0, D_OUT_PAD - D_OUT)))
    ws2 = jnp.pad(W_self2, ((0, 0), (0, D_OUT_PAD - D_OUT)))
    b2p = jnp.pad(b2, (0, D_OUT_PAD - D_OUT)).reshape(1, D_OUT_PAD)

    deg_parts = _make_deg()(ones128, zeros128, dst_p)
    deg3 = deg_parts.reshape(NC, ACC_ROWS, DEG_W)

    agg128 = _make_agg(D_HID)
    agg64 = _make_agg(D_OUT_PAD)

    p0, s0 = _tc_first(x, W_neigh0, W_self0, b0.reshape(1, D_HID))
    a0 = agg128(p0, src_p, dst_p, zeros128).reshape(NC, ACC_ROWS, D_HID)

    p1, s1 = _tc_mid(s0, a0, deg3, W_neigh1, W_self1,
                     b1.reshape(1, D_HID), D_HID)
    a1 = agg128(p1, src_p, dst_p, zeros128).reshape(NC, ACC_ROWS, D_HID)

    p2, s2 = _tc_mid(s1, a1, deg3, wn2, ws2, b2p, D_OUT_PAD)
    a2 = agg64(p2, src_p, dst_p, zeros64).reshape(NC, ACC_ROWS, D_OUT_PAD)

    out = _tc_last(s2, a2, deg3)
    return out[:, :D_OUT]
